# baseline (speedup 1.0000x reference)
import functools

import jax
import jax.numpy as jnp
from jax import lax
from jax.experimental import pallas as pl
from jax.experimental.pallas import tpu as pltpu
from jax.experimental.pallas import tpu_sc as plsc

B, L, D, F, T = 8, 512, 256, 256, 1536
NW = 32
RPW = (B * T) // NW
CHUNK = 128
NCH = RPW // CHUNK
LN_EPS = 1e-5


def _tc_body(mel_ref, bl_ref, x_ref, tgt_ref,
             w1a, w1b, w1c, bc1_r, g1_r, b1_r,
             w2a, w2b, w2c, bc2_r, g2_r, b2_r,
             wl_r, dur_ref, idx_ref):
    i = pl.program_id(0)
    xb = x_ref[0]
    zrow = jnp.zeros((1, D), jnp.float32)
    xm1 = jnp.concatenate([zrow, xb[:-1, :]], axis=0)
    xp1 = jnp.concatenate([xb[1:, :], zrow], axis=0)

    def ln_relu(h, g, b):
        m = jnp.mean(h, axis=-1, keepdims=True)
        v = jnp.mean((h - m) * (h - m), axis=-1, keepdims=True)
        hn = (h - m) * lax.rsqrt(v + LN_EPS)
        return jnp.maximum(hn * g + b, 0.0)

    h = (jnp.dot(xm1, w1a[...]) + jnp.dot(xb, w1b[...]) + jnp.dot(xp1, w1c[...])
         + bc1_r[...])
    h = ln_relu(h, g1_r[...], b1_r[...])
    hm1 = jnp.concatenate([zrow, h[:-1, :]], axis=0)
    hp1 = jnp.concatenate([h[1:, :], zrow], axis=0)
    h2 = (jnp.dot(hm1, w2a[...]) + jnp.dot(h, w2b[...]) + jnp.dot(hp1, w2c[...])
          + bc2_r[...])
    h2 = ln_relu(h2, g2_r[...], b2_r[...])
    dur = jnp.sum(h2 * wl_r[...], axis=-1, keepdims=True) + bl_ref[0]
    dur_ref[0] = jnp.maximum(dur, 0.0)

    tgt_row = tgt_ref[0]
    ia = lax.broadcasted_iota(jnp.int32, (L, L), 0)
    ib = lax.broadcasted_iota(jnp.int32, (L, L), 1)
    m_tri = (ia <= ib).astype(jnp.float32)
    ends_row = jax.lax.dot(tgt_row, m_tri,
                           precision=jax.lax.Precision.HIGHEST)

    t_col = lax.broadcasted_iota(jnp.int32, (T, 1), 0)
    cmp = (ends_row <= t_col.astype(jnp.float32)).astype(jnp.float32)
    idx_i = jnp.sum(cmp, axis=-1, keepdims=True).astype(jnp.int32)
    valid = (idx_i < L) & (t_col < mel_ref[0])
    idx_ref[0] = jnp.where(valid, i * L + idx_i, B * L)


def _sc_gather_body(xpad_hbm, idx_hbm, out_hbm, idx_v, rows_v, sem):
    wid = lax.axis_index("s") * 2 + lax.axis_index("c")
    pltpu.sync_copy(idx_hbm.at[wid], idx_v)
    copies = []
    for k in range(NCH):
        copies.append(pltpu.async_copy(
            xpad_hbm.at[idx_v.at[k]],
            rows_v.at[pl.ds(k * CHUNK, CHUNK)], sem))
    for c in copies:
        c.wait()
    pltpu.sync_copy(rows_v, out_hbm.at[pl.ds(wid * RPW, RPW)])


def _make_sc_gather():
    return pl.kernel(
        _sc_gather_body,
        out_type=jax.ShapeDtypeStruct((B * T, D), jnp.float32),
        mesh=plsc.VectorSubcoreMesh(core_axis_name="c", subcore_axis_name="s"),
        scratch_types=[
            pltpu.VMEM((NCH, CHUNK), jnp.int32),
            pltpu.VMEM((RPW, D), jnp.float32),
            pltpu.SemaphoreType.DMA,
        ],
    )


def kernel(x, target, mel_max_length, Wc1, bc1, g1, b1, Wc2, bc2, g2, b2, Wl, bl):
    x = x.astype(jnp.float32)
    tgt3 = target.astype(jnp.float32).reshape(B, 1, L)
    mel = jnp.asarray(mel_max_length, jnp.int32).reshape(1)
    blv = bl.astype(jnp.float32).reshape(1)

    row = lambda a: a.astype(jnp.float32).reshape(1, F)
    w1 = [jnp.transpose(Wc1[:, :, k]) for k in range(3)]
    w2 = [jnp.transpose(Wc2[:, :, k]) for k in range(3)]
    wl_row = Wl.astype(jnp.float32).reshape(1, F)

    full = lambda shp: pl.BlockSpec(shp, lambda i: (0,) * len(shp))
    dur3, idx3 = pl.pallas_call(
        _tc_body,
        grid=(B,),
        in_specs=[
            pl.BlockSpec(memory_space=pltpu.SMEM),
            pl.BlockSpec(memory_space=pltpu.SMEM),
            pl.BlockSpec((1, L, D), lambda i: (i, 0, 0)),
            pl.BlockSpec((1, 1, L), lambda i: (i, 0, 0)),
            full((D, F)), full((D, F)), full((D, F)),
            full((1, F)), full((1, F)), full((1, F)),
            full((F, F)), full((F, F)), full((F, F)),
            full((1, F)), full((1, F)), full((1, F)),
            full((1, F)),
        ],
        out_specs=[
            pl.BlockSpec((1, L, 1), lambda i: (i, 0, 0)),
            pl.BlockSpec((1, T, 1), lambda i: (i, 0, 0)),
        ],
        out_shape=[
            jax.ShapeDtypeStruct((B, L, 1), jnp.float32),
            jax.ShapeDtypeStruct((B, T, 1), jnp.int32),
        ],
    )(mel, blv, x, tgt3,
      w1[0], w1[1], w1[2], row(bc1), row(g1), row(b1),
      w2[0], w2[1], w2[2], row(bc2), row(g2), row(b2),
      wl_row)

    dur = dur3.reshape(B, L)
    xpad = jnp.concatenate(
        [x.reshape(B * L, D), jnp.zeros((8, D), jnp.float32)], axis=0)
    idx_w = idx3.reshape(NW, NCH, CHUNK)
    out = _make_sc_gather()(xpad, idx_w).reshape(B, T, D)
    return (out, dur)


# speedup vs baseline: 1.0145x; 1.0145x over previous
import jax
import jax.numpy as jnp
from jax import lax
from jax.experimental import pallas as pl
from jax.experimental.pallas import tpu as pltpu
from jax.experimental.pallas import tpu_sc as plsc

B, L, D, F, T = 8, 512, 256, 256, 1536
BL = B * L
NW = 32
RPW = (B * T) // NW
CHUNK = 32
NCH = RPW // CHUNK
HB = B // 2
SCROWS = HB * L + 8
ZROW = HB * L
LN_EPS = 1e-5


def _tc_body(mel_ref, bl_ref, x_ref, tgt_ref,
             w1a, w1b, w1c, bc1_r, g1_r, b1_r,
             w2a, w2b, w2c, bc2_r, g2_r, b2_r,
             wl_r, dur_ref, idx_ref, ends_ref):
    xb = x_ref[...]
    r = lax.broadcasted_iota(jnp.int32, (BL, 1), 0)
    first = (r % L == 0)
    last = (r % L == (L - 1))
    zrow = jnp.zeros((1, D), jnp.float32)

    def shift_up(a):
        s = jnp.concatenate([zrow, a[:-1, :]], axis=0)
        return jnp.where(first, 0.0, s)

    def shift_dn(a):
        s = jnp.concatenate([a[1:, :], zrow], axis=0)
        return jnp.where(last, 0.0, s)

    def ln_relu(h, g, b):
        m = jnp.mean(h, axis=-1, keepdims=True)
        v = jnp.mean((h - m) * (h - m), axis=-1, keepdims=True)
        hn = (h - m) * lax.rsqrt(v + LN_EPS)
        return jnp.maximum(hn * g + b, 0.0)

    def mm(a, w):
        return jnp.dot(a.astype(jnp.bfloat16), w[...],
                       preferred_element_type=jnp.float32)

    h = mm(shift_up(xb), w1a) + mm(xb, w1b) + mm(shift_dn(xb), w1c) + bc1_r[...]
    h = ln_relu(h, g1_r[...], b1_r[...])
    h2 = mm(shift_up(h), w2a) + mm(h, w2b) + mm(shift_dn(h), w2c) + bc2_r[...]
    h2 = ln_relu(h2, g2_r[...], b2_r[...])
    dur = jnp.sum(h2 * wl_r[...], axis=-1, keepdims=True) + bl_ref[0]
    dur_ref[...] = jnp.maximum(dur, 0.0)

    ia = lax.broadcasted_iota(jnp.int32, (L, L), 0)
    ib = lax.broadcasted_iota(jnp.int32, (L, L), 1)
    m_tri = (ia <= ib).astype(jnp.float32)
    ends_ref[...] = jax.lax.dot(tgt_ref[...], m_tri,
                                precision=jax.lax.Precision.HIGHEST)

    t_col = lax.broadcasted_iota(jnp.int32, (T, 1), 0)
    t_f = t_col.astype(jnp.float32)
    mel = mel_ref[0]

    def per_batch(b, _):
        ends_row = ends_ref[pl.ds(b, 1), :]
        cmp = (ends_row <= t_f).astype(jnp.float32)
        idx_i = jnp.sum(cmp, axis=-1, keepdims=True).astype(jnp.int32)
        valid = (idx_i < L) & (t_col < mel)
        local = jnp.where(valid, b * L + idx_i, BL)
        idx_ref[pl.ds(b * T, T), :] = local
        return 0

    lax.fori_loop(0, B, per_batch, 0)


def _sc_gather_body(xpad_hbm, idx_hbm, out_hbm, idx_v, rows_v, sem):
    c = lax.axis_index("c")
    s = lax.axis_index("s")
    wid = c * 16 + s
    pltpu.sync_copy(idx_hbm.at[wid], idx_v)
    copies = []
    for k in range(NCH):
        copies.append(pltpu.async_copy(
            xpad_hbm.at[idx_v.at[k]],
            rows_v.at[pl.ds(k * CHUNK, CHUNK)], sem))
    for cp in copies:
        cp.wait()
    pltpu.sync_copy(rows_v, out_hbm.at[pl.ds(wid * RPW, RPW)])


def _make_sc_gather():
    return pl.kernel(
        _sc_gather_body,
        out_type=jax.ShapeDtypeStruct((B * T, D), jnp.float32),
        mesh=plsc.VectorSubcoreMesh(core_axis_name="c", subcore_axis_name="s"),
        scratch_types=[
            pltpu.VMEM((NCH, CHUNK), jnp.int32),
            pltpu.VMEM((RPW, D), jnp.float32),
            pltpu.SemaphoreType.DMA,
        ],
    )


def kernel(x, target, mel_max_length, Wc1, bc1, g1, b1, Wc2, bc2, g2, b2, Wl, bl):
    x = x.astype(jnp.float32)
    xf = x.reshape(BL, D)
    tgt = target.astype(jnp.float32).reshape(B, L)
    mel = jnp.asarray(mel_max_length, jnp.int32).reshape(1)
    blv = bl.astype(jnp.float32).reshape(1)

    row = lambda a: a.astype(jnp.float32).reshape(1, F)
    wmat = lambda W, k: jnp.transpose(W[:, :, k]).astype(jnp.bfloat16)
    wl_row = Wl.astype(jnp.float32).reshape(1, F)

    full = lambda shp: pl.BlockSpec(shp, lambda: (0,) * len(shp))
    dur2, idx2 = pl.pallas_call(
        _tc_body,
        in_specs=[
            pl.BlockSpec(memory_space=pltpu.SMEM),
            pl.BlockSpec(memory_space=pltpu.SMEM),
            full((BL, D)), full((B, L)),
            full((D, F)), full((D, F)), full((D, F)),
            full((1, F)), full((1, F)), full((1, F)),
            full((F, F)), full((F, F)), full((F, F)),
            full((1, F)), full((1, F)), full((1, F)),
            full((1, F)),
        ],
        out_specs=[
            full((BL, 1)),
            full((B * T, 1)),
        ],
        out_shape=[
            jax.ShapeDtypeStruct((BL, 1), jnp.float32),
            jax.ShapeDtypeStruct((B * T, 1), jnp.int32),
        ],
        scratch_shapes=[pltpu.VMEM((B, L), jnp.float32)],
    )(mel, blv, xf, tgt,
      wmat(Wc1, 0), wmat(Wc1, 1), wmat(Wc1, 2), row(bc1), row(g1), row(b1),
      wmat(Wc2, 0), wmat(Wc2, 1), wmat(Wc2, 2), row(bc2), row(g2), row(b2),
      wl_row)

    dur = dur2.reshape(B, L)
    xpad = jnp.concatenate(
        [x.reshape(BL, D), jnp.zeros((8, D), jnp.float32)], axis=0)
    idx_w = idx2.reshape(NW, NCH, CHUNK)
    out = _make_sc_gather()(xpad, idx_w).reshape(B, T, D)
    return (out, dur)


# speedup vs baseline: 8.5534x; 8.4309x over previous
import jax
import jax.numpy as jnp
from jax import lax
from jax.experimental import pallas as pl
from jax.experimental.pallas import tpu as pltpu

B, L, D, F, T = 8, 512, 256, 256, 1536
BL = B * L
LN_EPS = 1e-5


def _predictor_body(mel_ref, bl_ref, x_ref, tgt_ref,
                    w1a, w1b, w1c, bc1_r, g1_r, b1_r,
                    w2a, w2b, w2c, bc2_r, g2_r, b2_r,
                    wl_r, dur_ref, starts_ref, ends_ref):
    xb = x_ref[...]
    r = lax.broadcasted_iota(jnp.int32, (BL, 1), 0)
    first = (r % L == 0)
    last = (r % L == (L - 1))
    zrow = jnp.zeros((1, D), jnp.float32)

    def shift_up(a):
        s = jnp.concatenate([zrow, a[:-1, :]], axis=0)
        return jnp.where(first, 0.0, s)

    def shift_dn(a):
        s = jnp.concatenate([a[1:, :], zrow], axis=0)
        return jnp.where(last, 0.0, s)

    def ln_relu(h, g, b):
        m = jnp.mean(h, axis=-1, keepdims=True)
        v = jnp.mean((h - m) * (h - m), axis=-1, keepdims=True)
        hn = (h - m) * lax.rsqrt(v + LN_EPS)
        return jnp.maximum(hn * g + b, 0.0)

    def mm(a, w):
        return jnp.dot(a.astype(jnp.bfloat16), w[...],
                       preferred_element_type=jnp.float32)

    h = mm(shift_up(xb), w1a) + mm(xb, w1b) + mm(shift_dn(xb), w1c) + bc1_r[...]
    h = ln_relu(h, g1_r[...], b1_r[...])
    h2 = mm(shift_up(h), w2a) + mm(h, w2b) + mm(shift_dn(h), w2c) + bc2_r[...]
    h2 = ln_relu(h2, g2_r[...], b2_r[...])
    dur = jnp.sum(h2 * wl_r[...], axis=-1, keepdims=True) + bl_ref[0]
    dur_ref[...] = jnp.maximum(dur, 0.0)

    ia = lax.broadcasted_iota(jnp.int32, (L, L), 0)
    ib = lax.broadcasted_iota(jnp.int32, (L, L), 1)
    m_tri = (ia <= ib).astype(jnp.float32)
    tgt = tgt_ref[...]
    ends = jax.lax.dot(tgt, m_tri, precision=jax.lax.Precision.HIGHEST)
    starts_ref[...] = ends - tgt
    ends_ref[...] = jnp.minimum(ends, mel_ref[0].astype(jnp.float32))


def _align_body(x_ref, starts_ref, ends_ref, out_ref):
    t_f = lax.broadcasted_iota(jnp.int32, (T, 1), 0).astype(jnp.float32)
    starts_row = starts_ref[0]
    ends_row = ends_ref[0]
    a = ((t_f >= starts_row) & (t_f < ends_row)).astype(jnp.bfloat16)
    out_ref[0] = jnp.dot(a, x_ref[0].astype(jnp.bfloat16),
                         preferred_element_type=jnp.float32)


def kernel(x, target, mel_max_length, Wc1, bc1, g1, b1, Wc2, bc2, g2, b2, Wl, bl):
    x = x.astype(jnp.float32)
    tgt = target.astype(jnp.float32).reshape(B, L)
    mel = jnp.asarray(mel_max_length, jnp.int32).reshape(1)
    blv = bl.astype(jnp.float32).reshape(1)

    row = lambda a: a.astype(jnp.float32).reshape(1, F)
    wmat = lambda W, k: jnp.transpose(W[:, :, k]).astype(jnp.bfloat16)

    full = lambda shp: pl.BlockSpec(shp, lambda: (0,) * len(shp))
    dur2, starts, ends = pl.pallas_call(
        _predictor_body,
        in_specs=[
            pl.BlockSpec(memory_space=pltpu.SMEM),
            pl.BlockSpec(memory_space=pltpu.SMEM),
            full((BL, D)), full((B, L)),
            full((D, F)), full((D, F)), full((D, F)),
            full((1, F)), full((1, F)), full((1, F)),
            full((F, F)), full((F, F)), full((F, F)),
            full((1, F)), full((1, F)), full((1, F)),
            full((1, F)),
        ],
        out_specs=[full((BL, 1)), full((B, L)), full((B, L))],
        out_shape=[
            jax.ShapeDtypeStruct((BL, 1), jnp.float32),
            jax.ShapeDtypeStruct((B, L), jnp.float32),
            jax.ShapeDtypeStruct((B, L), jnp.float32),
        ],
    )(mel, blv, x.reshape(BL, D), tgt,
      wmat(Wc1, 0), wmat(Wc1, 1), wmat(Wc1, 2), row(bc1), row(g1), row(b1),
      wmat(Wc2, 0), wmat(Wc2, 1), wmat(Wc2, 2), row(bc2), row(g2), row(b2),
      Wl.astype(jnp.float32).reshape(1, F))

    out = pl.pallas_call(
        _align_body,
        grid=(B,),
        in_specs=[
            pl.BlockSpec((1, L, D), lambda i: (i, 0, 0)),
            pl.BlockSpec((1, 1, L), lambda i: (i, 0, 0)),
            pl.BlockSpec((1, 1, L), lambda i: (i, 0, 0)),
        ],
        out_specs=pl.BlockSpec((1, T, D), lambda i: (i, 0, 0)),
        out_shape=jax.ShapeDtypeStruct((B, T, D), jnp.float32),
    )(x, starts.reshape(B, 1, L), ends.reshape(B, 1, L))

    return (out, dur2.reshape(B, L))


# speedup vs baseline: 10.1357x; 1.1850x over previous
import jax
import jax.numpy as jnp
from jax import lax
from jax.experimental import pallas as pl
from jax.experimental.pallas import tpu as pltpu

B, L, D, F, T = 8, 512, 256, 256, 1536
LN_EPS = 1e-5


def _body(mel_ref, bl_ref, x_ref, tgt_ref,
          w1a, w1b, w1c, bc1_r, g1_r, b1_r,
          w2a, w2b, w2c, bc2_r, g2_r, b2_r,
          wl_r, out_ref, dur_ref):
    xb = x_ref[0]
    zrow = jnp.zeros((1, D), jnp.float32)

    def ln_relu(h, g, b):
        m = jnp.mean(h, axis=-1, keepdims=True)
        v = jnp.mean((h - m) * (h - m), axis=-1, keepdims=True)
        hn = (h - m) * lax.rsqrt(v + LN_EPS)
        return jnp.maximum(hn * g + b, 0.0)

    def conv(a, wu, wc, wd, bias):
        up = jnp.concatenate([zrow, a[:-1, :]], axis=0)
        dn = jnp.concatenate([a[1:, :], zrow], axis=0)
        mm = lambda t, w: jnp.dot(t.astype(jnp.bfloat16), w[...],
                                  preferred_element_type=jnp.float32)
        return mm(up, wu) + mm(a, wc) + mm(dn, wd) + bias[...]

    h = ln_relu(conv(xb, w1a, w1b, w1c, bc1_r), g1_r[...], b1_r[...])
    h2 = ln_relu(conv(h, w2a, w2b, w2c, bc2_r), g2_r[...], b2_r[...])
    dur = jnp.sum(h2 * wl_r[...], axis=-1, keepdims=True) + bl_ref[0]
    dur_ref[0] = jnp.maximum(dur, 0.0)

    ia = lax.broadcasted_iota(jnp.int32, (L, L), 0)
    ib = lax.broadcasted_iota(jnp.int32, (L, L), 1)
    m_tri = (ia <= ib).astype(jnp.float32)
    tgt_row = tgt_ref[0]
    ends_row = jax.lax.dot(tgt_row, m_tri, precision=jax.lax.Precision.HIGHEST)
    starts_row = ends_row - tgt_row
    ends_row = jnp.minimum(ends_row, mel_ref[0].astype(jnp.float32))

    t_f = lax.broadcasted_iota(jnp.int32, (T, 1), 0).astype(jnp.float32)
    a_mat = ((t_f >= starts_row) & (t_f < ends_row)).astype(jnp.bfloat16)
    out_ref[0] = jnp.dot(a_mat, xb.astype(jnp.bfloat16),
                         preferred_element_type=jnp.float32)


def kernel(x, target, mel_max_length, Wc1, bc1, g1, b1, Wc2, bc2, g2, b2, Wl, bl):
    x = x.astype(jnp.float32)
    tgt3 = target.astype(jnp.float32).reshape(B, 1, L)
    mel = jnp.asarray(mel_max_length, jnp.int32).reshape(1)
    blv = bl.astype(jnp.float32).reshape(1)

    row = lambda a: a.astype(jnp.float32).reshape(1, F)
    wmat = lambda W, k: jnp.transpose(W[:, :, k]).astype(jnp.bfloat16)

    full = lambda shp: pl.BlockSpec(shp, lambda i: (0,) * len(shp))
    out, dur3 = pl.pallas_call(
        _body,
        grid=(B,),
        in_specs=[
            pl.BlockSpec(memory_space=pltpu.SMEM),
            pl.BlockSpec(memory_space=pltpu.SMEM),
            pl.BlockSpec((1, L, D), lambda i: (i, 0, 0)),
            pl.BlockSpec((1, 1, L), lambda i: (i, 0, 0)),
            full((D, F)), full((D, F)), full((D, F)),
            full((1, F)), full((1, F)), full((1, F)),
            full((F, F)), full((F, F)), full((F, F)),
            full((1, F)), full((1, F)), full((1, F)),
            full((1, F)),
        ],
        out_specs=[
            pl.BlockSpec((1, T, D), lambda i: (i, 0, 0)),
            pl.BlockSpec((1, L, 1), lambda i: (i, 0, 0)),
        ],
        out_shape=[
            jax.ShapeDtypeStruct((B, T, D), jnp.float32),
            jax.ShapeDtypeStruct((B, L, 1), jnp.float32),
        ],
    )(mel, blv, x, tgt3,
      wmat(Wc1, 0), wmat(Wc1, 1), wmat(Wc1, 2), row(bc1), row(g1), row(b1),
      wmat(Wc2, 0), wmat(Wc2, 1), wmat(Wc2, 2), row(bc2), row(g2), row(b2),
      Wl.astype(jnp.float32).reshape(1, F))

    return (out, dur3.reshape(B, L))
